# baseline (speedup 1.0000x reference)
import jax
import jax.numpy as jnp
from jax import lax
from jax.experimental import pallas as pl
from jax.experimental.pallas import tpu as pltpu

_TM = 400
_NR = 2


def _body(adj_ref, x_ref, w1_ref, b1_ref, w2_ref, b2_ref, out_ref,
          s1_ref, s2_ref, rb_ref, abuf_ref, xbuf_ref, obuf_ref,
          sem_ref, xsem_ref, osem_ref):
    n = adj_ref.shape[0]
    nt = n // _TM
    nr = rb_ref.shape[0] // 2
    nfetch = 2 * nt - nr
    total = 2 * nt

    def _xcopy(c, slot):
        return pltpu.make_async_copy(
            x_ref.at[pl.ds(c * _TM, _TM), :], xbuf_ref.at[slot],
            xsem_ref.at[slot])

    _xcopy(0, 0).start()

    def _xloop(c, carry):
        slot = lax.rem(c, 2)

        @pl.when(c + 1 < nt)
        def _():
            _xcopy(c + 1, lax.rem(c + 1, 2)).start()

        _xcopy(c, slot).wait()
        s1_ref[pl.ds(c * _TM, _TM), :] = jnp.dot(
            xbuf_ref[slot].astype(jnp.bfloat16), w1_ref[...],
            preferred_element_type=jnp.float32).astype(jnp.bfloat16)
        return carry

    lax.fori_loop(0, nt, _xloop, 0)

    def _copy(k, slot):
        tile = lax.rem(k, nt)
        return pltpu.make_async_copy(
            adj_ref.at[pl.ds(tile * _TM, _TM), :], abuf_ref.at[slot],
            sem_ref.at[slot])

    def _ocopy(t):
        slot = lax.rem(t, 2)
        return pltpu.make_async_copy(
            obuf_ref.at[slot], out_ref.at[pl.ds(t * _TM, _TM), :],
            osem_ref.at[slot])

    _copy(0, 0).start()

    def _loop(i, carry):
        consuming = i < nfetch
        slot = lax.rem(i, 2)

        @pl.when(i + 1 < nfetch)
        def _():
            _copy(i + 1, lax.rem(i + 1, 2)).start()

        @pl.when(consuming)
        def _():
            _copy(i, slot).wait()

        chunks = [(c * 2048, min(2048, n - c * 2048)) for c in range(5)]

        @pl.when(i < nt)
        def _():
            acc = jnp.zeros((_TM, s1_ref.shape[1]), jnp.float32)
            for off, sz in chunks:
                ac = abuf_ref[slot, :, off:off + sz].astype(jnp.bfloat16)
                acc += jnp.dot(ac, s1_ref[pl.ds(off, sz), :],
                               preferred_element_type=jnp.float32)
            h = jnp.maximum(acc + b1_ref[...], 0.0).astype(jnp.bfloat16)
            s2_ref[pl.ds(i * _TM, _TM), :] = jnp.dot(
                h, w2_ref[...], preferred_element_type=jnp.float32
            ).astype(jnp.bfloat16)

            @pl.when(i >= nt - nr)
            def _():
                j = 2 * (i - (nt - nr))
                rb_ref[j] = abuf_ref[slot, : _TM // 2, :].astype(jnp.bfloat16)
                rb_ref[j + 1] = abuf_ref[slot, _TM // 2:, :].astype(
                    jnp.bfloat16)

        @pl.when(i >= nt)
        def _():
            t = i - nt
            oslot = lax.rem(t, 2)

            @pl.when(t >= 2)
            def _():
                _ocopy(t - 2).wait()

            @pl.when(consuming)
            def _():
                acc = jnp.zeros((_TM, s2_ref.shape[1]), jnp.float32)
                for off, sz in chunks:
                    ac = abuf_ref[slot, :, off:off + sz].astype(jnp.bfloat16)
                    acc += jnp.dot(ac, s2_ref[pl.ds(off, sz), :],
                                   preferred_element_type=jnp.float32)
                obuf_ref[oslot] = acc + b2_ref[...]

            @pl.when(~consuming)
            def _():
                j = 2 * (i - nfetch)
                acc0 = jnp.dot(rb_ref[j], s2_ref[...],
                               preferred_element_type=jnp.float32)
                acc1 = jnp.dot(rb_ref[j + 1], s2_ref[...],
                               preferred_element_type=jnp.float32)
                obuf_ref[oslot] = (
                    jnp.concatenate([acc0, acc1], axis=0) + b2_ref[...])

            _ocopy(t).start()

        return carry

    lax.fori_loop(0, total, _loop, 0)
    _ocopy(nt - 2).wait()
    _ocopy(nt - 1).wait()


def kernel(x, adj, W1, b1, W2, b2):
    n, nfeat = x.shape
    nhid = W1.shape[1]
    nout = W2.shape[1]
    w1b = W1.astype(jnp.bfloat16)
    w2b = W2.astype(jnp.bfloat16)
    b1r = b1.reshape(1, nhid)
    b2r = b2.reshape(1, nout)
    nt = n // _TM
    nr = min(_NR, nt - 1)

    out = pl.pallas_call(
        _body,
        in_specs=[
            pl.BlockSpec(memory_space=pl.ANY),
            pl.BlockSpec(memory_space=pl.ANY),
            pl.BlockSpec(memory_space=pltpu.VMEM),
            pl.BlockSpec(memory_space=pltpu.VMEM),
            pl.BlockSpec(memory_space=pltpu.VMEM),
            pl.BlockSpec(memory_space=pltpu.VMEM),
        ],
        out_specs=pl.BlockSpec(memory_space=pl.ANY),
        out_shape=jax.ShapeDtypeStruct((n, nout), jnp.float32),
        scratch_shapes=[
            pltpu.VMEM((n, nhid), jnp.bfloat16),
            pltpu.VMEM((n, nout), jnp.bfloat16),
            pltpu.VMEM((2 * nr, _TM // 2, n), jnp.bfloat16),
            pltpu.VMEM((2, _TM, n), jnp.float32),
            pltpu.VMEM((2, _TM, nfeat), jnp.float32),
            pltpu.VMEM((2, _TM, nout), jnp.float32),
            pltpu.SemaphoreType.DMA((2,)),
            pltpu.SemaphoreType.DMA((2,)),
            pltpu.SemaphoreType.DMA((2,)),
        ],
    )(adj, x, w1b, b1r, w2b, b2r)

    return out


# speedup vs baseline: 1.0250x; 1.0250x over previous
import jax
import jax.numpy as jnp
from jax import lax
from jax.experimental import pallas as pl
from jax.experimental.pallas import tpu as pltpu

_TM = 400


def _fused_body(x_ref, adj_ref, w1_ref, b1_ref, w2_ref, b2_ref, out_ref,
                s1_ref, s2_ref):
    i = pl.program_id(0)
    p = pl.num_programs(0) // 2

    @pl.when(i == 0)
    def _():
        s1_ref[...] = jnp.dot(
            x_ref[...].astype(jnp.bfloat16), w1_ref[...],
            preferred_element_type=jnp.float32).astype(jnp.bfloat16)

    a = adj_ref[...].astype(jnp.bfloat16)

    @pl.when(i < p)
    def _():
        acc = jnp.dot(a, s1_ref[...], preferred_element_type=jnp.float32)
        h = jnp.maximum(acc + b1_ref[...], 0.0).astype(jnp.bfloat16)
        s2_ref[pl.ds(i * _TM, _TM), :] = jnp.dot(
            h, w2_ref[...], preferred_element_type=jnp.float32
        ).astype(jnp.bfloat16)

    @pl.when(i >= p)
    def _():
        acc = jnp.dot(a, s2_ref[...], preferred_element_type=jnp.float32)
        out_ref[...] = acc + b2_ref[...]


def kernel(x, adj, W1, b1, W2, b2):
    n, nfeat = x.shape
    nhid = W1.shape[1]
    nout = W2.shape[1]
    w1b = W1.astype(jnp.bfloat16)
    w2b = W2.astype(jnp.bfloat16)
    b1r = b1.reshape(1, nhid)
    b2r = b2.reshape(1, nout)

    p = n // _TM
    grid = (2 * p,)

    out = pl.pallas_call(
        _fused_body,
        grid=grid,
        in_specs=[
            pl.BlockSpec((n, nfeat), lambda i: (0, 0)),
            pl.BlockSpec((_TM, n), lambda i: (
                jnp.where(i < p, i, jnp.where(i == p, p - 1, i - p - 1)), 0)),
            pl.BlockSpec((nfeat, nhid), lambda i: (0, 0)),
            pl.BlockSpec((1, nhid), lambda i: (0, 0)),
            pl.BlockSpec((nhid, nout), lambda i: (0, 0)),
            pl.BlockSpec((1, nout), lambda i: (0, 0)),
        ],
        out_specs=pl.BlockSpec((_TM, nout), lambda i: (
            jnp.where(i <= p, jnp.where(i < p, 0, p - 1), i - p - 1), 0)),
        out_shape=jax.ShapeDtypeStruct((n, nout), jnp.float32),
        scratch_shapes=[
            pltpu.VMEM((n, nhid), jnp.bfloat16),
            pltpu.VMEM((n, nout), jnp.bfloat16),
        ],
        compiler_params=pltpu.CompilerParams(
            dimension_semantics=("arbitrary",)),
    )(x, adj, w1b, b1r, w2b, b2r)

    return out


# speedup vs baseline: 1.0399x; 1.0146x over previous
import jax
import jax.numpy as jnp
from jax import lax
from jax.experimental import pallas as pl
from jax.experimental.pallas import tpu as pltpu

_TM = 400


def _fused_body(x_ref, adj_ref, w1_ref, b1_ref, w2_ref, b2_ref, out_ref,
                s1_ref, s2_ref):
    i = pl.program_id(0)
    p = pl.num_programs(0) // 2

    @pl.when(i == 0)
    def _():
        s1_ref[...] = jnp.dot(
            x_ref[...].astype(jnp.bfloat16), w1_ref[...],
            preferred_element_type=jnp.float32).astype(jnp.bfloat16)

    @pl.when(i < p)
    def _():
        a0 = adj_ref[:, :5120].astype(jnp.bfloat16)
        acc = jnp.dot(a0, s1_ref[pl.ds(0, 5120), :],
                      preferred_element_type=jnp.float32)
        a1 = adj_ref[:, 5120:].astype(jnp.bfloat16)
        acc = acc + jnp.dot(a1, s1_ref[pl.ds(5120, 4880), :],
                            preferred_element_type=jnp.float32)
        h = jnp.maximum(acc + b1_ref[...], 0.0).astype(jnp.bfloat16)
        s2_ref[pl.ds(i * _TM, _TM), :] = jnp.dot(
            h, w2_ref[...], preferred_element_type=jnp.float32
        ).astype(jnp.bfloat16)

    @pl.when(i >= p)
    def _():
        a = adj_ref[...].astype(jnp.bfloat16)
        acc = jnp.dot(a, s2_ref[...], preferred_element_type=jnp.float32)
        out_ref[...] = acc + b2_ref[...]


def kernel(x, adj, W1, b1, W2, b2):
    n, nfeat = x.shape
    nhid = W1.shape[1]
    nout = W2.shape[1]
    w1b = W1.astype(jnp.bfloat16)
    w2b = W2.astype(jnp.bfloat16)
    b1r = b1.reshape(1, nhid)
    b2r = b2.reshape(1, nout)

    p = n // _TM
    grid = (2 * p,)

    out = pl.pallas_call(
        _fused_body,
        grid=grid,
        in_specs=[
            pl.BlockSpec((n, nfeat), lambda i: (0, 0)),
            pl.BlockSpec((_TM, n), lambda i: (
                jnp.where(i < p, i, jnp.where(i == p, p - 1, i - p - 1)), 0)),
            pl.BlockSpec((nfeat, nhid), lambda i: (0, 0)),
            pl.BlockSpec((1, nhid), lambda i: (0, 0)),
            pl.BlockSpec((nhid, nout), lambda i: (0, 0)),
            pl.BlockSpec((1, nout), lambda i: (0, 0)),
        ],
        out_specs=pl.BlockSpec((_TM, nout), lambda i: (
            jnp.where(i <= p, jnp.where(i < p, 0, p - 1), i - p - 1), 0)),
        out_shape=jax.ShapeDtypeStruct((n, nout), jnp.float32),
        scratch_shapes=[
            pltpu.VMEM((n, nhid), jnp.bfloat16),
            pltpu.VMEM((n, nout), jnp.bfloat16),
        ],
        compiler_params=pltpu.CompilerParams(
            dimension_semantics=("arbitrary",)),
    )(x, adj, w1b, b1r, w2b, b2r)

    return out


# speedup vs baseline: 1.0400x; 1.0000x over previous
import jax
import jax.numpy as jnp
from jax import lax
from jax.experimental import pallas as pl
from jax.experimental.pallas import tpu as pltpu

_TM = 400


def _fused_body(x_ref, adj_ref, w1_ref, b1_ref, w2_ref, b2_ref, out_ref,
                s1_ref, s2_ref):
    i = pl.program_id(0)
    p = pl.num_programs(0) // 2

    @pl.when(i == 0)
    def _():
        s1_ref[...] = jnp.dot(
            x_ref[...].astype(jnp.bfloat16), w1_ref[...],
            preferred_element_type=jnp.float32).astype(jnp.bfloat16)

    @pl.when(i < p)
    def _():
        acc = 0.0
        for off, sz in ((0, 2560), (2560, 2560), (5120, 2560), (7680, 2320)):
            ac = adj_ref[:, off:off + sz].astype(jnp.bfloat16)
            acc = acc + jnp.dot(ac, s1_ref[pl.ds(off, sz), :],
                                preferred_element_type=jnp.float32)
        h = jnp.maximum(acc + b1_ref[...], 0.0).astype(jnp.bfloat16)
        s2_ref[pl.ds(i * _TM, _TM), :] = jnp.dot(
            h, w2_ref[...], preferred_element_type=jnp.float32
        ).astype(jnp.bfloat16)

    @pl.when(i >= p)
    def _():
        a = adj_ref[...].astype(jnp.bfloat16)
        acc = jnp.dot(a, s2_ref[...], preferred_element_type=jnp.float32)
        out_ref[...] = acc + b2_ref[...]


def kernel(x, adj, W1, b1, W2, b2):
    n, nfeat = x.shape
    nhid = W1.shape[1]
    nout = W2.shape[1]
    w1b = W1.astype(jnp.bfloat16)
    w2b = W2.astype(jnp.bfloat16)
    b1r = b1.reshape(1, nhid)
    b2r = b2.reshape(1, nout)

    p = n // _TM
    grid = (2 * p,)

    out = pl.pallas_call(
        _fused_body,
        grid=grid,
        in_specs=[
            pl.BlockSpec((n, nfeat), lambda i: (0, 0)),
            pl.BlockSpec((_TM, n), lambda i: (
                jnp.where(i < p, i, jnp.where(i == p, p - 1, i - p - 1)), 0)),
            pl.BlockSpec((nfeat, nhid), lambda i: (0, 0)),
            pl.BlockSpec((1, nhid), lambda i: (0, 0)),
            pl.BlockSpec((nhid, nout), lambda i: (0, 0)),
            pl.BlockSpec((1, nout), lambda i: (0, 0)),
        ],
        out_specs=pl.BlockSpec((_TM, nout), lambda i: (
            jnp.where(i <= p, jnp.where(i < p, 0, p - 1), i - p - 1), 0)),
        out_shape=jax.ShapeDtypeStruct((n, nout), jnp.float32),
        scratch_shapes=[
            pltpu.VMEM((n, nhid), jnp.bfloat16),
            pltpu.VMEM((n, nout), jnp.bfloat16),
        ],
        compiler_params=pltpu.CompilerParams(
            dimension_semantics=("arbitrary",)),
    )(x, adj, w1b, b1r, w2b, b2r)

    return out


# speedup vs baseline: 1.0417x; 1.0016x over previous
import jax
import jax.numpy as jnp
from jax import lax
from jax.experimental import pallas as pl
from jax.experimental.pallas import tpu as pltpu

_TM = 400


def _fused_body(x_ref, adj_ref, w1_ref, b1_ref, w2_ref, b2_ref, out_ref,
                s1_ref, s2_ref):
    i = pl.program_id(0)
    p = pl.num_programs(0) // 2

    @pl.when(i == 0)
    def _():
        s1_ref[...] = jnp.dot(
            x_ref[...].astype(jnp.bfloat16), w1_ref[...],
            preferred_element_type=jnp.float32).astype(jnp.bfloat16)

    @pl.when(i < p)
    def _():
        a0 = adj_ref[:, :5120].astype(jnp.bfloat16)
        acc = jnp.dot(a0, s1_ref[pl.ds(0, 5120), :],
                      preferred_element_type=jnp.float32)
        a1 = adj_ref[:, 5120:].astype(jnp.bfloat16)
        acc = acc + jnp.dot(a1, s1_ref[pl.ds(5120, 4880), :],
                            preferred_element_type=jnp.float32)
        h = jnp.maximum(acc + b1_ref[...], 0.0).astype(jnp.bfloat16)
        s2_ref[pl.ds(i * _TM, _TM), :] = jnp.dot(
            h, w2_ref[...], preferred_element_type=jnp.float32
        ).astype(jnp.bfloat16)

    @pl.when(i >= p)
    def _():
        a0 = adj_ref[:, :5120].astype(jnp.bfloat16)
        acc = jnp.dot(a0, s2_ref[pl.ds(0, 5120), :],
                      preferred_element_type=jnp.float32)
        a1 = adj_ref[:, 5120:].astype(jnp.bfloat16)
        acc = acc + jnp.dot(a1, s2_ref[pl.ds(5120, 4880), :],
                            preferred_element_type=jnp.float32)
        out_ref[...] = acc + b2_ref[...]


def kernel(x, adj, W1, b1, W2, b2):
    n, nfeat = x.shape
    nhid = W1.shape[1]
    nout = W2.shape[1]
    w1b = W1.astype(jnp.bfloat16)
    w2b = W2.astype(jnp.bfloat16)
    b1r = b1.reshape(1, nhid)
    b2r = b2.reshape(1, nout)

    p = n // _TM
    grid = (2 * p,)

    out = pl.pallas_call(
        _fused_body,
        grid=grid,
        in_specs=[
            pl.BlockSpec((n, nfeat), lambda i: (0, 0)),
            pl.BlockSpec((_TM, n), lambda i: (
                jnp.where(i < p, i, jnp.where(i == p, p - 1, i - p - 1)), 0)),
            pl.BlockSpec((nfeat, nhid), lambda i: (0, 0)),
            pl.BlockSpec((1, nhid), lambda i: (0, 0)),
            pl.BlockSpec((nhid, nout), lambda i: (0, 0)),
            pl.BlockSpec((1, nout), lambda i: (0, 0)),
        ],
        out_specs=pl.BlockSpec((_TM, nout), lambda i: (
            jnp.where(i <= p, jnp.where(i < p, 0, p - 1), i - p - 1), 0)),
        out_shape=jax.ShapeDtypeStruct((n, nout), jnp.float32),
        scratch_shapes=[
            pltpu.VMEM((n, nhid), jnp.bfloat16),
            pltpu.VMEM((n, nout), jnp.bfloat16),
        ],
        compiler_params=pltpu.CompilerParams(
            dimension_semantics=("arbitrary",)),
    )(x, adj, w1b, b1r, w2b, b2r)

    return out


# speedup vs baseline: 1.0420x; 1.0003x over previous
import jax
import jax.numpy as jnp
from jax import lax
from jax.experimental import pallas as pl
from jax.experimental.pallas import tpu as pltpu

_TM = 400


def _fused_body(x_ref, adj_ref, w1_ref, b1_ref, w2_ref, b2_ref, out_ref,
                s1_ref, s2_ref):
    i = pl.program_id(0)
    p = pl.num_programs(0) // 2

    @pl.when(i == 0)
    def _():
        s1_ref[...] = jnp.dot(
            x_ref[...].astype(jnp.bfloat16), w1_ref[...],
            preferred_element_type=jnp.float32).astype(jnp.bfloat16)

    @pl.when(i < p)
    def _():
        a0 = adj_ref[:, :5120].astype(jnp.bfloat16)
        acc = jnp.dot(a0, s1_ref[pl.ds(0, 5120), :],
                      preferred_element_type=jnp.float32)
        a1 = adj_ref[:, 5120:].astype(jnp.bfloat16)
        acc = acc + jnp.dot(a1, s1_ref[pl.ds(5120, 4880), :],
                            preferred_element_type=jnp.float32)
        h = jnp.maximum(acc + b1_ref[...], 0.0).astype(jnp.bfloat16)
        s2_ref[pl.ds(i * _TM, _TM), :] = jnp.dot(
            h, w2_ref[...], preferred_element_type=jnp.float32
        ).astype(jnp.bfloat16)

    @pl.when(i >= p)
    def _():
        a = adj_ref[...].astype(jnp.bfloat16)
        acc = jnp.dot(a, s2_ref[...], preferred_element_type=jnp.float32)
        out_ref[...] = acc + b2_ref[...]


def kernel(x, adj, W1, b1, W2, b2):
    n, nfeat = x.shape
    nhid = W1.shape[1]
    nout = W2.shape[1]
    w1b = W1.astype(jnp.bfloat16)
    w2b = W2.astype(jnp.bfloat16)
    b1r = b1.reshape(1, nhid)
    b2r = b2.reshape(1, nout)

    p = n // _TM
    grid = (2 * p,)

    out = pl.pallas_call(
        _fused_body,
        grid=grid,
        in_specs=[
            pl.BlockSpec((n, nfeat), lambda i: (0, 0)),
            pl.BlockSpec((_TM, n), lambda i: (
                jnp.where(i < p, i, jnp.where(i == p, p - 1, i - p - 1)), 0)),
            pl.BlockSpec((nfeat, nhid), lambda i: (0, 0)),
            pl.BlockSpec((1, nhid), lambda i: (0, 0)),
            pl.BlockSpec((nhid, nout), lambda i: (0, 0)),
            pl.BlockSpec((1, nout), lambda i: (0, 0)),
        ],
        out_specs=pl.BlockSpec((_TM, nout), lambda i: (
            jnp.where(i <= p, jnp.where(i < p, 0, p - 1), i - p - 1), 0)),
        out_shape=jax.ShapeDtypeStruct((n, nout), jnp.float32),
        scratch_shapes=[
            pltpu.VMEM((n, nhid), jnp.bfloat16),
            pltpu.VMEM((n, nout), jnp.bfloat16),
        ],
        compiler_params=pltpu.CompilerParams(
            dimension_semantics=("arbitrary",)),
    )(x, adj, w1b, b1r, w2b, b2r)

    return out
